# Initial kernel scaffold; baseline (speedup 1.0000x reference)
import jax
import jax.numpy as jnp
from jax.experimental import pallas as pl


def kernel(x, W1, b1, W2, b2, We1, be1, We2, be2):
    raise NotImplementedError("write your pallas kernel here")


# speedup vs baseline: 11.8344x; 11.8344x over previous
import jax
import jax.numpy as jnp
from jax import lax
from jax.experimental import pallas as pl
from jax.experimental.pallas import tpu as pltpu

_PROMPT_K = 16
_TOPK = 2
_H_BLK = 1024


def _gelu(v):
    return 0.5 * v * (1.0 + lax.erf(v * 0.7071067811865476))


def _router_kernel(x_ref, w1_ref, b1_ref, w2_ref, b2_ref,
                   probs_ref, stats_ref, emask_ref, twf_ref, idx_ref):
    n, c = x_ref.shape
    e = w2_ref.shape[0]
    x = x_ref[...]
    xm = jnp.sum(x, axis=0, keepdims=True) * (1.0 / n)
    num = jnp.sum(x * xm, axis=1, keepdims=True)
    xn = jnp.sqrt(jnp.sum(x * x, axis=1, keepdims=True))
    mn = jnp.sqrt(jnp.sum(xm * xm))
    eps = 1e-8
    sim = num / (jnp.maximum(xn, eps) * jnp.maximum(mn, eps))

    iota_n = lax.broadcasted_iota(jnp.int32, (n, 1), 0)

    def body(_, carry):
        simw, mask = carry
        m = jnp.max(simw)
        idx = jnp.min(jnp.where(simw == m, iota_n, n))
        hit = iota_n == idx
        mask = jnp.where(hit, 1.0, mask)
        simw = jnp.where(hit, -jnp.inf, simw)
        return simw, mask

    _, mask = lax.fori_loop(0, _PROMPT_K, body,
                            (sim, jnp.zeros((n, 1), jnp.float32)))
    context = jnp.sum(x * mask, axis=0, keepdims=True) * (1.0 / _PROMPT_K)

    h = _gelu(lax.dot_general(context, w1_ref[...],
                              (((1,), (1,)), ((), ())),
                              preferred_element_type=jnp.float32)
              + b1_ref[...])
    logits = lax.dot_general(h, w2_ref[...],
                             (((1,), (1,)), ((), ())),
                             preferred_element_type=jnp.float32) + b2_ref[...]
    ex = jnp.exp(logits - jnp.max(logits))
    p = ex / jnp.sum(ex)

    iota_e = lax.broadcasted_iota(jnp.int32, (1, e), 1)
    m1 = jnp.max(p)
    i1 = jnp.min(jnp.where(p == m1, iota_e, e))
    p2 = jnp.where(iota_e == i1, -1.0, p)
    m2 = jnp.max(p2)
    i2 = jnp.min(jnp.where(p2 == m2, iota_e, e))
    s = m1 + m2 + 1e-9
    tw0 = m1 / s
    tw1 = m2 / s

    local_ent = jnp.sum(-p * jnp.log(p + 1e-6))
    global_ent = jnp.sum(p * jnp.log(p + 1e-6))

    w2m = w2_ref[...]
    rn = jnp.sqrt(jnp.sum(w2m * w2m, axis=1, keepdims=True))
    wn = w2m / jnp.maximum(rn, 1e-12)
    simm = lax.dot_general(wn, wn, (((1,), (1,)), ((), ())),
                           preferred_element_type=jnp.float32)
    eye = (lax.broadcasted_iota(jnp.int32, (e, e), 0)
           == lax.broadcasted_iota(jnp.int32, (e, e), 1)).astype(jnp.float32)
    ortho = jnp.sqrt(jnp.sum((simm - eye) ** 2))

    hot1 = iota_e == i1
    hot2 = iota_e == i2
    probs_ref[...] = p
    emask_ref[...] = (hot1 | hot2).astype(jnp.float32)
    twf_ref[...] = jnp.where(hot1, tw0, 0.0) + jnp.where(hot2, tw1, 0.0)
    stats_ref[...] = (jnp.where(iota_e == 0, local_ent, 0.0)
                      + jnp.where(iota_e == 1, global_ent, 0.0)
                      + jnp.where(iota_e == 2, ortho, 0.0)
                      + jnp.where(iota_e == 3, tw0, 0.0)
                      + jnp.where(iota_e == 4, tw1, 0.0))
    idx_ref[...] = (jnp.where(iota_e == 0, i1, 0)
                    + jnp.where(iota_e == 1, i2, 0)).astype(jnp.int32)


def _expert_kernel(idx_ref, w_ref, x_ref,
                   we1a_ref, we1b_ref, be1a_ref, be1b_ref,
                   we2a_ref, we2b_ref, be2a_ref, be2b_ref, out_ref):
    h = pl.program_id(0)
    lane = lax.broadcasted_iota(jnp.int32, w_ref.shape, 1)
    w0 = jnp.sum(jnp.where(lane == 0, w_ref[...], 0.0))
    w1 = jnp.sum(jnp.where(lane == 1, w_ref[...], 0.0))

    x = x_ref[...]
    bf = jnp.bfloat16
    f32 = jnp.float32
    hpa = jnp.dot(x, we1a_ref[0].astype(bf), preferred_element_type=f32)
    hpa = _gelu(hpa + be1a_ref[0]) * w0
    hpb = jnp.dot(x, we1b_ref[0].astype(bf), preferred_element_type=f32)
    hpb = _gelu(hpb + be1b_ref[0]) * w1
    o = (jnp.dot(hpa.astype(bf), we2a_ref[0].astype(bf),
                 preferred_element_type=f32)
         + jnp.dot(hpb.astype(bf), we2b_ref[0].astype(bf),
                   preferred_element_type=f32))

    @pl.when(h == 0)
    def _():
        out_ref[...] = (w0 * be2a_ref[0] + w1 * be2b_ref[0]) + o

    @pl.when(h != 0)
    def _():
        out_ref[...] += o


def kernel(x, W1, b1, W2, b2, We1, be1, We2, be2):
    b, n, c = x.shape
    e, _, hid = We1.shape
    xs = x.reshape(n, c)

    probs, stats, emask, twf, idx8 = pl.pallas_call(
        _router_kernel,
        out_shape=(
            jax.ShapeDtypeStruct((1, e), jnp.float32),
            jax.ShapeDtypeStruct((1, e), jnp.float32),
            jax.ShapeDtypeStruct((1, e), jnp.float32),
            jax.ShapeDtypeStruct((1, e), jnp.float32),
            jax.ShapeDtypeStruct((1, e), jnp.int32),
        ),
    )(xs, W1, b1.reshape(1, -1), W2, b2.reshape(1, -1))

    topk_i = idx8[:, :_TOPK]
    topk_w = stats[:, 3:3 + _TOPK]
    w_pad = jnp.pad(topk_w, ((0, 0), (0, 8 - _TOPK)))

    ht = hid // _H_BLK
    grid_spec = pltpu.PrefetchScalarGridSpec(
        num_scalar_prefetch=1,
        grid=(ht,),
        in_specs=[
            pl.BlockSpec((1, 8), lambda h, idx: (0, 0)),
            pl.BlockSpec((n, c), lambda h, idx: (0, 0)),
            pl.BlockSpec((1, c, _H_BLK), lambda h, idx: (idx[0], 0, h)),
            pl.BlockSpec((1, c, _H_BLK), lambda h, idx: (idx[1], 0, h)),
            pl.BlockSpec((1, 1, _H_BLK), lambda h, idx: (idx[0], 0, h)),
            pl.BlockSpec((1, 1, _H_BLK), lambda h, idx: (idx[1], 0, h)),
            pl.BlockSpec((1, _H_BLK, c), lambda h, idx: (idx[0], h, 0)),
            pl.BlockSpec((1, _H_BLK, c), lambda h, idx: (idx[1], h, 0)),
            pl.BlockSpec((1, 1, c), lambda h, idx: (idx[0], 0, 0)),
            pl.BlockSpec((1, 1, c), lambda h, idx: (idx[1], 0, 0)),
        ],
        out_specs=pl.BlockSpec((n, c), lambda h, idx: (0, 0)),
    )
    be1r = be1.reshape(e, 1, hid)
    be2r = be2.reshape(e, 1, c)
    out = pl.pallas_call(
        _expert_kernel,
        grid_spec=grid_spec,
        out_shape=jax.ShapeDtypeStruct((n, c), jnp.float32),
    )(topk_i.reshape(-1), w_pad, xs.astype(jnp.bfloat16),
      We1, We1, be1r, be1r, We2, We2, be2r, be2r)

    output = out.reshape(b, n, c)
    local_ent = stats[0, 0]
    global_ent = stats[0, 1]
    ortho = stats[0, 2]
    expert_mask_mean = emask[0]
    return (output, local_ent, global_ent, ortho, expert_mask_mean,
            probs, topk_i, twf)


# speedup vs baseline: 12.3989x; 1.0477x over previous
import jax
import jax.numpy as jnp
from jax import lax
from jax.experimental import pallas as pl
from jax.experimental.pallas import tpu as pltpu

_PROMPT_K = 16
_TOPK = 2
_H_BLK = 1024


def _gelu(v):
    return 0.5 * v * (1.0 + lax.erf(v * 0.7071067811865476))


def _router_kernel(x_ref, w1_ref, b1_ref, w2_ref, b2_ref,
                   probs_ref, stats_ref, emask_ref, twf_ref, idx_ref,
                   xbf_ref):
    n, c = x_ref.shape
    e = w2_ref.shape[0]
    x = x_ref[...]
    xbf_ref[...] = x.astype(jnp.bfloat16)
    xm = jnp.sum(x, axis=0, keepdims=True) * (1.0 / n)
    num = jnp.sum(x * xm, axis=1, keepdims=True)
    xn = jnp.sqrt(jnp.sum(x * x, axis=1, keepdims=True))
    mn = jnp.sqrt(jnp.sum(xm * xm))
    eps = 1e-8
    sim = num / (jnp.maximum(xn, eps) * jnp.maximum(mn, eps))

    iota_n = lax.broadcasted_iota(jnp.int32, (n, 1), 0)

    def body(_, carry):
        simw, mask = carry
        m = jnp.max(simw)
        idx = jnp.min(jnp.where(simw == m, iota_n, n))
        hit = iota_n == idx
        mask = jnp.where(hit, 1.0, mask)
        simw = jnp.where(hit, -jnp.inf, simw)
        return simw, mask

    _, mask = lax.fori_loop(0, _PROMPT_K, body,
                            (sim, jnp.zeros((n, 1), jnp.float32)))
    context = jnp.sum(x * mask, axis=0, keepdims=True) * (1.0 / _PROMPT_K)

    h = _gelu(lax.dot_general(context, w1_ref[...],
                              (((1,), (1,)), ((), ())),
                              preferred_element_type=jnp.float32)
              + b1_ref[...])
    logits = lax.dot_general(h, w2_ref[...],
                             (((1,), (1,)), ((), ())),
                             preferred_element_type=jnp.float32) + b2_ref[...]
    ex = jnp.exp(logits - jnp.max(logits))
    p = ex / jnp.sum(ex)

    iota_e = lax.broadcasted_iota(jnp.int32, (1, e), 1)
    m1 = jnp.max(p)
    i1 = jnp.min(jnp.where(p == m1, iota_e, e))
    p2 = jnp.where(iota_e == i1, -1.0, p)
    m2 = jnp.max(p2)
    i2 = jnp.min(jnp.where(p2 == m2, iota_e, e))
    s = m1 + m2 + 1e-9
    tw0 = m1 / s
    tw1 = m2 / s

    local_ent = jnp.sum(-p * jnp.log(p + 1e-6))
    global_ent = jnp.sum(p * jnp.log(p + 1e-6))

    w2m = w2_ref[...]
    rn = jnp.sqrt(jnp.sum(w2m * w2m, axis=1, keepdims=True))
    wn = w2m / jnp.maximum(rn, 1e-12)
    simm = lax.dot_general(wn, wn, (((1,), (1,)), ((), ())),
                           preferred_element_type=jnp.float32)
    eye = (lax.broadcasted_iota(jnp.int32, (e, e), 0)
           == lax.broadcasted_iota(jnp.int32, (e, e), 1)).astype(jnp.float32)
    ortho = jnp.sqrt(jnp.sum((simm - eye) ** 2))

    hot1 = iota_e == i1
    hot2 = iota_e == i2
    probs_ref[...] = p
    emask_ref[...] = (hot1 | hot2).astype(jnp.float32)
    twf_ref[...] = jnp.where(hot1, tw0, 0.0) + jnp.where(hot2, tw1, 0.0)
    stats_ref[...] = (jnp.where(iota_e == 0, local_ent, 0.0)
                      + jnp.where(iota_e == 1, global_ent, 0.0)
                      + jnp.where(iota_e == 2, ortho, 0.0)
                      + jnp.where(iota_e == 3, tw0, 0.0)
                      + jnp.where(iota_e == 4, tw1, 0.0))
    idx_ref[...] = (jnp.where(iota_e == 0, i1, 0)
                    + jnp.where(iota_e == 1, i2, 0)).astype(jnp.int32)


def _expert_kernel(idx_ref, w_ref, x_ref,
                   we1a_ref, we1b_ref, be1a_ref, be1b_ref,
                   we2a_ref, we2b_ref, be2a_ref, be2b_ref, out_ref):
    h = pl.program_id(0)
    lane = lax.broadcasted_iota(jnp.int32, w_ref.shape, 1)
    w0 = jnp.sum(jnp.where(lane == 0, w_ref[...], 0.0))
    w1 = jnp.sum(jnp.where(lane == 1, w_ref[...], 0.0))

    x = x_ref[...]
    bf = jnp.bfloat16
    f32 = jnp.float32
    hpa = jnp.dot(x, we1a_ref[0].astype(bf), preferred_element_type=f32)
    hpa = _gelu(hpa + be1a_ref[0]) * w0
    hpb = jnp.dot(x, we1b_ref[0].astype(bf), preferred_element_type=f32)
    hpb = _gelu(hpb + be1b_ref[0]) * w1
    o = (jnp.dot(hpa.astype(bf), we2a_ref[0].astype(bf),
                 preferred_element_type=f32)
         + jnp.dot(hpb.astype(bf), we2b_ref[0].astype(bf),
                   preferred_element_type=f32))

    @pl.when(h == 0)
    def _():
        out_ref[...] = (w0 * be2a_ref[0] + w1 * be2b_ref[0]) + o

    @pl.when(h != 0)
    def _():
        out_ref[...] += o


def kernel(x, W1, b1, W2, b2, We1, be1, We2, be2):
    b, n, c = x.shape
    e, _, hid = We1.shape
    xs = x.reshape(n, c)

    probs, stats, emask, twf, idx8, xbf = pl.pallas_call(
        _router_kernel,
        out_shape=(
            jax.ShapeDtypeStruct((1, e), jnp.float32),
            jax.ShapeDtypeStruct((1, e), jnp.float32),
            jax.ShapeDtypeStruct((1, e), jnp.float32),
            jax.ShapeDtypeStruct((1, e), jnp.float32),
            jax.ShapeDtypeStruct((1, e), jnp.int32),
            jax.ShapeDtypeStruct((n, c), jnp.bfloat16),
        ),
    )(xs, W1, b1.reshape(1, -1), W2, b2.reshape(1, -1))

    topk_i = idx8[:, :_TOPK]
    topk_w = stats[:, 3:3 + _TOPK]
    w_pad = jnp.pad(topk_w, ((0, 0), (0, 8 - _TOPK)))

    ht = hid // _H_BLK
    grid_spec = pltpu.PrefetchScalarGridSpec(
        num_scalar_prefetch=1,
        grid=(ht,),
        in_specs=[
            pl.BlockSpec((1, 8), lambda h, idx: (0, 0)),
            pl.BlockSpec((n, c), lambda h, idx: (0, 0)),
            pl.BlockSpec((1, c, _H_BLK), lambda h, idx: (idx[0], 0, h)),
            pl.BlockSpec((1, c, _H_BLK), lambda h, idx: (idx[1], 0, h)),
            pl.BlockSpec((1, 1, _H_BLK), lambda h, idx: (idx[0], 0, h)),
            pl.BlockSpec((1, 1, _H_BLK), lambda h, idx: (idx[1], 0, h)),
            pl.BlockSpec((1, _H_BLK, c), lambda h, idx: (idx[0], h, 0)),
            pl.BlockSpec((1, _H_BLK, c), lambda h, idx: (idx[1], h, 0)),
            pl.BlockSpec((1, 1, c), lambda h, idx: (idx[0], 0, 0)),
            pl.BlockSpec((1, 1, c), lambda h, idx: (idx[1], 0, 0)),
        ],
        out_specs=pl.BlockSpec((n, c), lambda h, idx: (0, 0)),
    )
    be1r = be1.reshape(e, 1, hid)
    be2r = be2.reshape(e, 1, c)
    out = pl.pallas_call(
        _expert_kernel,
        grid_spec=grid_spec,
        out_shape=jax.ShapeDtypeStruct((n, c), jnp.float32),
    )(topk_i.reshape(-1), w_pad, xbf,
      We1, We1, be1r, be1r, We2, We2, be2r, be2r)

    output = out.reshape(b, n, c)
    local_ent = stats[0, 0]
    global_ent = stats[0, 1]
    ortho = stats[0, 2]
    expert_mask_mean = emask[0]
    return (output, local_ent, global_ent, ortho, expert_mask_mean,
            probs, topk_i, twf)


# speedup vs baseline: 13.9416x; 1.1244x over previous
import jax
import jax.numpy as jnp
from jax import lax
from jax.experimental import pallas as pl
from jax.experimental.pallas import tpu as pltpu

_PROMPT_K = 16
_TOPK = 2
_H_BLK = 512


def _gelu(v):
    return 0.5 * v * (1.0 + lax.erf(v * 0.7071067811865476))


def _router_kernel(x_ref, w1_ref, b1_ref, w2_ref, b2_ref,
                   probs_ref, stats_ref, emask_ref, twf_ref, wvec_ref,
                   idx_ref, xbf_ref):
    n, c = x_ref.shape
    e = w2_ref.shape[0]
    x = x_ref[...]
    xbf_ref[...] = x.astype(jnp.bfloat16)
    xm = jnp.sum(x, axis=0, keepdims=True) * (1.0 / n)
    num = jnp.sum(x * xm, axis=1, keepdims=True)
    xn = jnp.sqrt(jnp.sum(x * x, axis=1, keepdims=True))
    mn = jnp.sqrt(jnp.sum(xm * xm))
    eps = 1e-8
    sim = num / (jnp.maximum(xn, eps) * jnp.maximum(mn, eps))

    iota_n = lax.broadcasted_iota(jnp.int32, (n, 1), 0)

    def body(_, carry):
        simw, mask = carry
        m = jnp.max(simw)
        idx = jnp.min(jnp.where(simw == m, iota_n, n))
        hit = iota_n == idx
        mask = jnp.where(hit, 1.0, mask)
        simw = jnp.where(hit, -jnp.inf, simw)
        return simw, mask

    _, mask = lax.fori_loop(0, _PROMPT_K, body,
                            (sim, jnp.zeros((n, 1), jnp.float32)))
    context = jnp.sum(x * mask, axis=0, keepdims=True) * (1.0 / _PROMPT_K)

    h = _gelu(lax.dot_general(context, w1_ref[...],
                              (((1,), (1,)), ((), ())),
                              preferred_element_type=jnp.float32)
              + b1_ref[...])
    logits = lax.dot_general(h, w2_ref[...],
                             (((1,), (1,)), ((), ())),
                             preferred_element_type=jnp.float32) + b2_ref[...]
    ex = jnp.exp(logits - jnp.max(logits))
    p = ex / jnp.sum(ex)

    iota_e = lax.broadcasted_iota(jnp.int32, (1, e), 1)
    m1 = jnp.max(p)
    i1 = jnp.min(jnp.where(p == m1, iota_e, e))
    p2 = jnp.where(iota_e == i1, -1.0, p)
    m2 = jnp.max(p2)
    i2 = jnp.min(jnp.where(p2 == m2, iota_e, e))
    s = m1 + m2 + 1e-9
    tw0 = m1 / s
    tw1 = m2 / s

    local_ent = jnp.sum(-p * jnp.log(p + 1e-6))
    global_ent = jnp.sum(p * jnp.log(p + 1e-6))

    w2m = w2_ref[...]
    rn = jnp.sqrt(jnp.sum(w2m * w2m, axis=1, keepdims=True))
    wn = w2m / jnp.maximum(rn, 1e-12)
    simm = lax.dot_general(wn, wn, (((1,), (1,)), ((), ())),
                           preferred_element_type=jnp.float32)
    eye = (lax.broadcasted_iota(jnp.int32, (e, e), 0)
           == lax.broadcasted_iota(jnp.int32, (e, e), 1)).astype(jnp.float32)
    ortho = jnp.sqrt(jnp.sum((simm - eye) ** 2))

    hot1 = iota_e == i1
    hot2 = iota_e == i2
    probs_ref[...] = p
    emask_ref[...] = (hot1 | hot2).astype(jnp.float32)
    twf_ref[...] = jnp.where(hot1, tw0, 0.0) + jnp.where(hot2, tw1, 0.0)
    stats_ref[...] = (jnp.where(iota_e == 0, local_ent, 0.0)
                      + jnp.where(iota_e == 1, global_ent, 0.0)
                      + jnp.where(iota_e == 2, ortho, 0.0))
    wvec_ref[...] = (jnp.where(iota_e == 0, tw0, 0.0)
                     + jnp.where(iota_e == 1, tw1, 0.0))
    idx_ref[...] = (jnp.where(iota_e == 0, i1, 0)
                    + jnp.where(iota_e == 1, i2, 0)).astype(jnp.int32)


def _expert_kernel(idx_ref, w_ref, x_ref,
                   we1a_ref, we1b_ref, be1a_ref, be1b_ref,
                   we2a_ref, we2b_ref, be2a_ref, be2b_ref, out_ref):
    h = pl.program_id(0)
    lane = lax.broadcasted_iota(jnp.int32, w_ref.shape, 1)
    w0 = jnp.sum(jnp.where(lane == 0, w_ref[...], 0.0))
    w1 = jnp.sum(jnp.where(lane == 1, w_ref[...], 0.0))

    x = x_ref[...]
    bf = jnp.bfloat16
    f32 = jnp.float32
    hpa = jnp.dot(x, we1a_ref[0].astype(bf), preferred_element_type=f32)
    ga = _gelu((hpa + be1a_ref[0]).astype(bf))
    hpb = jnp.dot(x, we1b_ref[0].astype(bf), preferred_element_type=f32)
    gb = _gelu((hpb + be1b_ref[0]).astype(bf))
    o = (w0 * jnp.dot(ga, we2a_ref[0].astype(bf), preferred_element_type=f32)
         + w1 * jnp.dot(gb, we2b_ref[0].astype(bf),
                        preferred_element_type=f32))

    @pl.when(h == 0)
    def _():
        out_ref[...] = (w0 * be2a_ref[0] + w1 * be2b_ref[0]) + o

    @pl.when(h != 0)
    def _():
        out_ref[...] += o


def kernel(x, W1, b1, W2, b2, We1, be1, We2, be2):
    b, n, c = x.shape
    e, _, hid = We1.shape
    xs = x.reshape(n, c)

    probs, stats, emask, twf, wvec, idx8, xbf = pl.pallas_call(
        _router_kernel,
        out_shape=(
            jax.ShapeDtypeStruct((1, e), jnp.float32),
            jax.ShapeDtypeStruct((1, e), jnp.float32),
            jax.ShapeDtypeStruct((1, e), jnp.float32),
            jax.ShapeDtypeStruct((1, e), jnp.float32),
            jax.ShapeDtypeStruct((1, e), jnp.float32),
            jax.ShapeDtypeStruct((1, e), jnp.int32),
            jax.ShapeDtypeStruct((n, c), jnp.bfloat16),
        ),
    )(xs, W1, b1.reshape(1, -1), W2, b2.reshape(1, -1))

    topk_i = idx8[:, :_TOPK]

    ht = hid // _H_BLK
    grid_spec = pltpu.PrefetchScalarGridSpec(
        num_scalar_prefetch=1,
        grid=(ht,),
        in_specs=[
            pl.BlockSpec((1, 8), lambda h, idx: (0, 0)),
            pl.BlockSpec((n, c), lambda h, idx: (0, 0)),
            pl.BlockSpec((1, c, _H_BLK), lambda h, idx: (idx[0], 0, h)),
            pl.BlockSpec((1, c, _H_BLK), lambda h, idx: (idx[1], 0, h)),
            pl.BlockSpec((1, 1, _H_BLK), lambda h, idx: (idx[0], 0, h)),
            pl.BlockSpec((1, 1, _H_BLK), lambda h, idx: (idx[1], 0, h)),
            pl.BlockSpec((1, _H_BLK, c), lambda h, idx: (idx[0], h, 0)),
            pl.BlockSpec((1, _H_BLK, c), lambda h, idx: (idx[1], h, 0)),
            pl.BlockSpec((1, 1, c), lambda h, idx: (idx[0], 0, 0)),
            pl.BlockSpec((1, 1, c), lambda h, idx: (idx[1], 0, 0)),
        ],
        out_specs=pl.BlockSpec((n, c), lambda h, idx: (0, 0)),
    )
    be1r = be1.reshape(e, 1, hid)
    be2r = be2.reshape(e, 1, c)
    out = pl.pallas_call(
        _expert_kernel,
        grid_spec=grid_spec,
        out_shape=jax.ShapeDtypeStruct((n, c), jnp.float32),
    )(idx8.reshape(-1), wvec, xbf,
      We1, We1, be1r, be1r, We2, We2, be2r, be2r)

    output = out.reshape(b, n, c)
    local_ent = stats[0, 0]
    global_ent = stats[0, 1]
    ortho = stats[0, 2]
    expert_mask_mean = emask[0]
    return (output, local_ent, global_ent, ortho, expert_mask_mean,
            probs, topk_i, twf)
